# baseline (speedup 1.0000x reference)
import functools

import jax
import jax.numpy as jnp
from jax import lax
from jax.experimental import pallas as pl
from jax.experimental.pallas import tpu as pltpu
from jax.experimental.pallas import tpu_sc as plsc

B = 16
L = 512
H = 16
V = 64

NC = 2
NS = 16
NW = NC * NS

ROWS = B * L
RPW = ROWS // NW
R = 4
NCHUNK = RPW // R
NPAIR = NCHUNK // 2


def _sc_body(tbl_hbm, data_hbm, out_hbm, tbl_v, idx0, idx1, out0, out1,
             sem_i0, sem_i1, sem_o0, sem_o1):
    wid = lax.axis_index("s") * NC + lax.axis_index("c")
    pltpu.sync_copy(tbl_hbm, tbl_v)

    idx_v = (idx0, idx1)
    out_v = (out0, out1)
    sem_i = (sem_i0, sem_i1)
    sem_o = (sem_o0, sem_o1)

    b = wid // 2
    i_base = (wid % 2) * RPW

    pltpu.async_copy(data_hbm.at[b, pl.ds(i_base, R)], idx0, sem_i0)

    def process(c, slot):
        pltpu.make_async_copy(
            data_hbm.at[0, pl.ds(0, R)], idx_v[slot], sem_i[slot]).wait()

        @pl.when(c + 1 < NCHUNK)
        def _():
            pltpu.async_copy(
                data_hbm.at[b, pl.ds(i_base + (c + 1) * R, R)],
                idx_v[1 - slot], sem_i[1 - slot])

        @pl.when(c >= 2)
        def _():
            for h in range(H):
                pltpu.make_async_copy(
                    out_v[slot].at[h], out_hbm.at[0, h, pl.ds(0, R)],
                    sem_o[slot]).wait()

        for r in range(R):
            @plsc.parallel_loop(0, L, step=16, unroll=2)
            def group_body(col):
                idxv = idx_v[slot][r, pl.ds(col, 16)]
                for h in range(H):
                    out_v[slot][h, r, pl.ds(col, 16)] = plsc.load_gather(
                        tbl_v.at[pl.ds(h * V, V)], [idxv])

        i0 = i_base + c * R
        for h in range(H):
            pltpu.async_copy(
                out_v[slot].at[h], out_hbm.at[b, h, pl.ds(i0, R)],
                sem_o[slot])

    def pair_body(p, carry):
        process(2 * p, 0)
        process(2 * p + 1, 1)
        return carry

    lax.fori_loop(0, NPAIR, pair_body, 0)

    for slot in range(2):
        for h in range(H):
            pltpu.make_async_copy(
                out_v[slot].at[h], out_hbm.at[0, h, pl.ds(0, R)],
                sem_o[slot]).wait()


@jax.jit
def _hop_bias_sc(tbl, data):
    mesh = plsc.VectorSubcoreMesh(core_axis_name="c", subcore_axis_name="s")
    run = pl.kernel(
        _sc_body,
        out_type=jax.ShapeDtypeStruct((B, H, L, L), jnp.float32),
        mesh=mesh,
        scratch_types=[
            pltpu.VMEM((V * H,), jnp.float32),
            pltpu.VMEM((R, L), jnp.int32),
            pltpu.VMEM((R, L), jnp.int32),
            pltpu.VMEM((H, R, L), jnp.float32),
            pltpu.VMEM((H, R, L), jnp.float32),
            pltpu.SemaphoreType.DMA,
            pltpu.SemaphoreType.DMA,
            pltpu.SemaphoreType.DMA,
            pltpu.SemaphoreType.DMA,
        ],
        compiler_params=pltpu.CompilerParams(
            needs_layout_passes=False, use_tc_tiling_on_sc=False),
    )
    return run(tbl, data)


def kernel(data, hop_embeddings):
    return _hop_bias_sc(hop_embeddings.T.reshape(-1),
                        data.astype(jnp.int32))


# speedup vs baseline: 2.3664x; 2.3664x over previous
import functools

import jax
import jax.numpy as jnp
from jax import lax
from jax.experimental import pallas as pl
from jax.experimental.pallas import tpu as pltpu
from jax.experimental.pallas import tpu_sc as plsc

B = 16
L = 512
H = 16
V = 64

NC = 2
NS = 16
NW = NC * NS

ROWS = B * L
RPW = ROWS // NW
R = 4
NCHUNK = RPW // R
NPAIR = NCHUNK // 2


def _sc_body(tbl_hbm, data_hbm, out_hbm, tbl_v, idx0, idx1, out0, out1,
             sem_i0, sem_i1, sem_o0, sem_o1):
    wid = lax.axis_index("s") * NC + lax.axis_index("c")
    pltpu.sync_copy(tbl_hbm, tbl_v)

    idx_v = (idx0, idx1)
    out_v = (out0, out1)
    sem_i = (sem_i0, sem_i1)
    sem_o = (sem_o0, sem_o1)

    b = wid // 2
    i_base = (wid % 2) * RPW

    pltpu.async_copy(data_hbm.at[b, pl.ds(i_base, R)], idx0, sem_i0)

    def process(c, slot):
        pltpu.make_async_copy(
            data_hbm.at[0, pl.ds(0, R)], idx_v[slot], sem_i[slot]).wait()

        @pl.when(c + 1 < NCHUNK)
        def _():
            pltpu.async_copy(
                data_hbm.at[b, pl.ds(i_base + (c + 1) * R, R)],
                idx_v[1 - slot], sem_i[1 - slot])

        @pl.when(c >= 2)
        def _():
            for h in range(H):
                pltpu.make_async_copy(
                    out_v[slot].at[h], out_hbm.at[0, h, pl.ds(0, R)],
                    sem_o[slot]).wait()

        for r in range(R):
            @plsc.parallel_loop(0, L, step=16, unroll=2)
            def group_body(col):
                idxv = idx_v[slot][r, pl.ds(col, 16)]
                for h in range(H):
                    out_v[slot][h, r, pl.ds(col, 16)] = plsc.load_gather(
                        tbl_v.at[pl.ds(h * V, V)], [idxv])

        i0 = i_base + c * R
        for h in range(H):
            pltpu.async_copy(
                out_v[slot].at[h], out_hbm.at[b, h, pl.ds(i0, R)],
                sem_o[slot])

    def pair_body(p, carry):
        process(2 * p, 0)
        process(2 * p + 1, 1)
        return carry

    lax.fori_loop(0, NPAIR, pair_body, 0)

    for slot in range(2):
        for h in range(H):
            pltpu.make_async_copy(
                out_v[slot].at[h], out_hbm.at[0, h, pl.ds(0, R)],
                sem_o[slot]).wait()


@jax.jit
def _hop_bias_sc(tbl, data):
    mesh = plsc.VectorSubcoreMesh(core_axis_name="c", subcore_axis_name="s")
    run = pl.kernel(
        _sc_body,
        out_type=jax.ShapeDtypeStruct((B, H, L, L), jnp.float32),
        mesh=mesh,
        scratch_types=[
            pltpu.VMEM((V * H,), jnp.float32),
            pltpu.VMEM((R, L), jnp.int32),
            pltpu.VMEM((R, L), jnp.int32),
            pltpu.VMEM((H, R, L), jnp.float32),
            pltpu.VMEM((H, R, L), jnp.float32),
            pltpu.SemaphoreType.DMA,
            pltpu.SemaphoreType.DMA,
            pltpu.SemaphoreType.DMA,
            pltpu.SemaphoreType.DMA,
        ],
        compiler_params=pltpu.CompilerParams(
            needs_layout_passes=False, use_tc_tiling_on_sc=True),
    )
    return run(tbl, data)


def kernel(data, hop_embeddings):
    return _hop_bias_sc(hop_embeddings.T.reshape(-1),
                        data.astype(jnp.int32))


# speedup vs baseline: 2.9264x; 1.2367x over previous
import functools

import jax
import jax.numpy as jnp
from jax import lax
from jax.experimental import pallas as pl
from jax.experimental.pallas import tpu as pltpu
from jax.experimental.pallas import tpu_sc as plsc

B = 16
L = 512
H = 16
V = 64
HH = H // 2

NC = 2
NS = 16
NW = NC * NS

ROWS = B * L
RPW = ROWS // NW
R = 8
NCHUNK = RPW // R
NPAIR = NCHUNK // 2


def _sc_body(tbl_hbm, data_hbm, out_hbm, tbl_v, idx0, idx1, bufa, bufb,
             sem_i0, sem_i1, sem_a, sem_b):
    wid = lax.axis_index("s") * NC + lax.axis_index("c")
    pltpu.sync_copy(tbl_hbm, tbl_v)

    idx_v = (idx0, idx1)

    b = wid // 2
    i_base = (wid % 2) * RPW

    pltpu.async_copy(data_hbm.at[b, pl.ds(i_base, R)], idx0, sem_i0)

    def gather_half(idx_slot, buf, h0):
        def r_body(r, carry):
            @plsc.parallel_loop(0, L, step=16, unroll=2)
            def group_body(col):
                idxv = idx_v[idx_slot][r, pl.ds(col, 16)]
                for h in range(HH):
                    buf[h, r, pl.ds(col, 16)] = plsc.load_gather(
                        tbl_v.at[pl.ds((h0 + h) * V, V)], [idxv])
            return carry

        lax.fori_loop(0, R, r_body, 0)

    def process(c, idx_slot):
        pltpu.make_async_copy(
            data_hbm.at[0, pl.ds(0, R)], idx_v[idx_slot],
            (sem_i0, sem_i1)[idx_slot]).wait()

        @pl.when(c + 1 < NCHUNK)
        def _():
            pltpu.async_copy(
                data_hbm.at[b, pl.ds(i_base + (c + 1) * R, R)],
                idx_v[1 - idx_slot], (sem_i0, sem_i1)[1 - idx_slot])

        i0 = i_base + c * R

        @pl.when(c >= 1)
        def _():
            for h in range(HH):
                pltpu.make_async_copy(
                    bufa.at[h], out_hbm.at[0, h, pl.ds(0, R)], sem_a).wait()

        gather_half(idx_slot, bufa, 0)
        for h in range(HH):
            pltpu.async_copy(
                bufa.at[h], out_hbm.at[b, h, pl.ds(i0, R)], sem_a)

        @pl.when(c >= 1)
        def _():
            for h in range(HH):
                pltpu.make_async_copy(
                    bufb.at[h], out_hbm.at[0, h, pl.ds(0, R)], sem_b).wait()

        gather_half(idx_slot, bufb, HH)
        for h in range(HH):
            pltpu.async_copy(
                bufb.at[h], out_hbm.at[b, HH + h, pl.ds(i0, R)], sem_b)

    def pair_body(p, carry):
        process(2 * p, 0)
        process(2 * p + 1, 1)
        return carry

    lax.fori_loop(0, NPAIR, pair_body, 0)

    for h in range(HH):
        pltpu.make_async_copy(
            bufa.at[h], out_hbm.at[0, h, pl.ds(0, R)], sem_a).wait()
        pltpu.make_async_copy(
            bufb.at[h], out_hbm.at[0, h, pl.ds(0, R)], sem_b).wait()


@jax.jit
def _hop_bias_sc(tbl, data):
    mesh = plsc.VectorSubcoreMesh(core_axis_name="c", subcore_axis_name="s")
    run = pl.kernel(
        _sc_body,
        out_type=jax.ShapeDtypeStruct((B, H, L, L), jnp.float32),
        mesh=mesh,
        scratch_types=[
            pltpu.VMEM((V * H,), jnp.float32),
            pltpu.VMEM((R, L), jnp.int32),
            pltpu.VMEM((R, L), jnp.int32),
            pltpu.VMEM((HH, R, L), jnp.float32),
            pltpu.VMEM((HH, R, L), jnp.float32),
            pltpu.SemaphoreType.DMA,
            pltpu.SemaphoreType.DMA,
            pltpu.SemaphoreType.DMA,
            pltpu.SemaphoreType.DMA,
        ],
        compiler_params=pltpu.CompilerParams(
            needs_layout_passes=False, use_tc_tiling_on_sc=True),
    )
    return run(tbl, data)


def kernel(data, hop_embeddings):
    return _hop_bias_sc(hop_embeddings.T.reshape(-1),
                        data.astype(jnp.int32))
